# baseline (speedup 1.0000x reference)
import functools

import jax
import jax.numpy as jnp
from jax import lax
from jax.experimental import pallas as pl
from jax.experimental.pallas import tpu as pltpu
from jax.experimental.pallas import tpu_sc as plsc

_VOCAB = 1000000
_EMBED = 64
_B = 4096
_N = 200

_NC = 2
_NS = 16
_NW = _NC * _NS
_WPG = 4
_NG = _NW // _WPG
_CPG = _N // _NG
_QB = _B // _WPG
_CH = 128
_JC = _QB // _CH

_mesh = plsc.VectorSubcoreMesh(core_axis_name="c", subcore_axis_name="s")


@functools.partial(
    pl.kernel,
    mesh=_mesh,
    compiler_params=pltpu.CompilerParams(use_tc_tiling_on_sc=False),
    out_type=jax.ShapeDtypeStruct((_B, _N, _EMBED), jnp.float32),
    scratch_types=[
        pltpu.VMEM((_CPG, _JC, _CH), jnp.int32),
        pltpu.VMEM((2, _CH, _EMBED), jnp.float32),
        pltpu.VMEM((_N, _EMBED), jnp.float32),
        pltpu.SemaphoreType.DMA,
        pltpu.SemaphoreType.DMA,
    ],
)
def _embed_sc(xT_hbm, tok_hbm, pos_hbm, out_hbm, idx_v, rows_v, pos_v, gsem, osem):
    cid = lax.axis_index("c")
    sid = lax.axis_index("s")
    wid = sid * _NC + cid
    grp = wid // _WPG
    sub = wid % _WPG
    n0 = grp * _CPG
    b0 = sub * _QB

    pltpu.sync_copy(pos_hbm, pos_v)
    pltpu.sync_copy(xT_hbm.at[pl.ds(n0, _CPG), pl.ds(sub * _JC, _JC)], idx_v)

    def t_body(t, carry):
        n = n0 + t
        prow = [pos_v[n, pl.ds(16 * d, 16)] for d in range(4)]
        for j in range(_JC):
            buf = j % 2
            pltpu.async_copy(tok_hbm.at[idx_v.at[t, j]], rows_v.at[buf], gsem).wait()

            def add_body(i, c, _buf=buf, _prow=prow):
                for ii in range(4):
                    row = i * 4 + ii
                    for d in range(4):
                        sl = pl.ds(16 * d, 16)
                        rows_v[_buf, row, sl] = rows_v[_buf, row, sl] + _prow[d]
                return c

            lax.fori_loop(0, _CH // 4, add_body, 0)
            pltpu.async_copy(
                rows_v.at[buf], out_hbm.at[pl.ds(b0 + j * _CH, _CH), n], osem
            ).wait()
        return carry

    lax.fori_loop(0, _CPG, t_body, 0)


def kernel(x, tok_table, pos_table):
    xT3 = x.astype(jnp.int32).T.reshape(_N, _B // _CH, _CH)
    return _embed_sc(xT3, tok_table, pos_table)


# speedup vs baseline: 1.1408x; 1.1408x over previous
import functools

import jax
import jax.numpy as jnp
from jax import lax
from jax.experimental import pallas as pl
from jax.experimental.pallas import tpu as pltpu
from jax.experimental.pallas import tpu_sc as plsc

_VOCAB = 1000000
_EMBED = 64
_B = 4096
_N = 200

_NC = 2
_NS = 16
_NW = _NC * _NS
_WPG = 4
_NG = _NW // _WPG
_CPG = _N // _NG
_QB = _B // _WPG
_CH = 128
_JC = _QB // _CH

_mesh = plsc.VectorSubcoreMesh(core_axis_name="c", subcore_axis_name="s")


@functools.partial(
    pl.kernel,
    mesh=_mesh,
    compiler_params=pltpu.CompilerParams(use_tc_tiling_on_sc=False),
    out_type=jax.ShapeDtypeStruct((_B, _N, _EMBED), jnp.float32),
    scratch_types=[
        pltpu.VMEM((_CPG, _JC, _CH), jnp.int32),
        pltpu.VMEM((2, _CH, _EMBED), jnp.float32),
        pltpu.VMEM((2, _CH, _EMBED), jnp.float32),
        pltpu.VMEM((_N, _EMBED), jnp.float32),
        pltpu.SemaphoreType.DMA,
        pltpu.SemaphoreType.DMA,
        pltpu.SemaphoreType.DMA,
        pltpu.SemaphoreType.DMA,
    ],
)
def _embed_sc(xT_hbm, tok_hbm, pos_hbm, out_hbm, idx_v, grows_v, orows_v, pos_v,
              gsem0, gsem1, osem0, osem1):
    cid = lax.axis_index("c")
    sid = lax.axis_index("s")
    wid = sid * _NC + cid
    grp = wid // _WPG
    sub = wid % _WPG
    n0 = grp * _CPG
    b0 = sub * _QB

    pltpu.sync_copy(pos_hbm, pos_v)
    pltpu.sync_copy(xT_hbm.at[pl.ds(n0, _CPG), pl.ds(sub * _JC, _JC)], idx_v)

    def gsem(b):
        return gsem0 if b == 0 else gsem1

    def osem(b):
        return osem0 if b == 0 else osem1

    def fire_gather(t, j):
        b = j % 2
        pltpu.async_copy(tok_hbm.at[idx_v.at[t, j]], grows_v.at[b], gsem(b))

    def out_slice(t, j):
        return out_hbm.at[pl.ds(b0 + j * _CH, _CH), n0 + t]

    def slot(t, j, do_outwait, do_fire):
        b = j % 2
        pltpu.make_async_copy(
            tok_hbm.at[idx_v.at[t, j]], grows_v.at[b], gsem(b)
        ).wait()
        if do_outwait:
            j3 = (j - 2) % _JC
            t3 = t - 1 if j < 2 else t
            pltpu.make_async_copy(orows_v.at[b], out_slice(t3, j3), osem(b)).wait()
        n = n0 + t
        prow = [pos_v[n, pl.ds(16 * d, 16)] for d in range(4)]

        def add_body(i, c):
            for ii in range(8):
                row = i * 8 + ii
                for d in range(4):
                    sl = pl.ds(16 * d, 16)
                    orows_v[b, row, sl] = grows_v[b, row, sl] + prow[d]
            return c

        lax.fori_loop(0, _CH // 8, add_body, 0)
        pltpu.async_copy(orows_v.at[b], out_slice(t, j), osem(b))
        if do_fire:
            j2 = (j + 2) % _JC
            t2 = t + 1 if j >= _JC - 2 else t
            fire_gather(t2, j2)

    fire_gather(0, 0)
    fire_gather(0, 1)

    for j in range(_JC):
        slot(0, j, do_outwait=(j >= 2), do_fire=True)

    def t_body(t, carry):
        for j in range(_JC):
            slot(t, j, do_outwait=True, do_fire=True)
        return carry

    lax.fori_loop(1, _CPG - 1, t_body, 0)

    for j in range(_JC):
        slot(_CPG - 1, j, do_outwait=True, do_fire=(j < _JC - 2))

    pltpu.make_async_copy(
        orows_v.at[0], out_slice(_CPG - 1, _JC - 2), osem0
    ).wait()
    pltpu.make_async_copy(
        orows_v.at[1], out_slice(_CPG - 1, _JC - 1), osem1
    ).wait()


def kernel(x, tok_table, pos_table):
    xT3 = x.astype(jnp.int32).T.reshape(_N, _B // _CH, _CH)
    return _embed_sc(xT3, tok_table, pos_table)
